# baseline (speedup 1.0000x reference)
import functools

import jax
import jax.numpy as jnp
from jax.experimental import pallas as pl

H = 8
D = 32
C = 256
L = 2

_BM = 1000


def _mm_body(x_ref, w_ref, b_ref, o_ref, *, act):
    acc = jnp.dot(x_ref[...], w_ref[...], preferred_element_type=jnp.float32)
    acc = acc + b_ref[...]
    if act == "relu":
        acc = jnp.maximum(acc, 0.0)
    o_ref[...] = acc


def _mm(x, w, b, act="none"):
    M, Cin = x.shape
    K = w.shape[1]
    grid = (M // _BM,)
    return pl.pallas_call(
        functools.partial(_mm_body, act=act),
        grid=grid,
        in_specs=[
            pl.BlockSpec((_BM, Cin), lambda i: (i, 0)),
            pl.BlockSpec((Cin, K), lambda i: (0, 0)),
            pl.BlockSpec((1, K), lambda i: (0, 0)),
        ],
        out_specs=pl.BlockSpec((_BM, K), lambda i: (i, 0)),
        out_shape=jax.ShapeDtypeStruct((M, K), jnp.float32),
    )(x, w, b.reshape(1, K))


def _out_body(agg_ref, x_ref, wa_ref, ba_ref, sk_ref, o_ref):
    g = jax.nn.gelu(agg_ref[...])
    acc = jnp.dot(g, wa_ref[...], preferred_element_type=jnp.float32)
    acc = acc + ba_ref[...]
    a = jax.nn.sigmoid(sk_ref[0, 0])
    o_ref[...] = a * acc + (1.0 - a) * x_ref[...]


def _out_proj(agg, x, wa, ba, skip):
    M = x.shape[0]
    grid = (M // _BM,)
    return pl.pallas_call(
        _out_body,
        grid=grid,
        in_specs=[
            pl.BlockSpec((_BM, C), lambda i: (i, 0)),
            pl.BlockSpec((_BM, C), lambda i: (i, 0)),
            pl.BlockSpec((C, C), lambda i: (0, 0)),
            pl.BlockSpec((1, C), lambda i: (0, 0)),
            pl.BlockSpec((1, 1), lambda i: (0, 0)),
        ],
        out_specs=pl.BlockSpec((_BM, C), lambda i: (i, 0)),
        out_shape=jax.ShapeDtypeStruct((M, C), jnp.float32),
    )(agg, x, wa, ba.reshape(1, C), skip.reshape(1, 1))


def _seg_softmax(att, dst, n):
    amax = jax.ops.segment_max(att, dst, num_segments=n)
    amax = jnp.where(jnp.isfinite(amax), amax, 0.0)
    ex = jnp.exp(att - amax[dst])
    den = jax.ops.segment_sum(ex, dst, num_segments=n)
    return ex / (den[dst] + 1e-16)


def kernel(x_paper, x_author, edge_index_writes, edge_index_rev, W_in, b_in,
           Wk, bk, Wq, bq, Wv, bv, Wa, ba, a_rel, m_rel, p_rel, skip):
    x = [_mm(x_paper, W_in[0], b_in[0], act="relu"),
         _mm(x_author, W_in[1], b_in[1], act="relu")]
    edges = [(1, 0, edge_index_writes), (0, 1, edge_index_rev)]
    for l in range(L):
        k = [_mm(x[t], Wk[l, t], bk[l, t]).reshape(-1, H, D) for t in range(2)]
        q = [_mm(x[t], Wq[l, t], bq[l, t]).reshape(-1, H, D) for t in range(2)]
        v = [_mm(x[t], Wv[l, t], bv[l, t]).reshape(-1, H, D) for t in range(2)]
        agg = [jnp.zeros((x[0].shape[0], H, D), jnp.float32),
               jnp.zeros((x[1].shape[0], H, D), jnp.float32)]
        for r, (st, dt, ei) in enumerate(edges):
            src, dst = ei[0], ei[1]
            n_dst = x[dt].shape[0]
            k_r = jnp.einsum('nhd,hde->nhe', k[st], a_rel[l, r])
            m_r = jnp.einsum('nhd,hde->nhe', v[st], m_rel[l, r])
            att = (q[dt][dst] * k_r[src]).sum(-1) * p_rel[l, r] / jnp.sqrt(float(D))
            alpha = _seg_softmax(att, dst, n_dst)
            msg = m_r[src] * alpha[:, :, None]
            agg[dt] = agg[dt] + jax.ops.segment_sum(msg, dst, num_segments=n_dst)
        x = [_out_proj(agg[t].reshape(-1, C), x[t], Wa[l, t], ba[l, t], skip[l, t])
             for t in range(2)]
    return x[0], x[1]
